# baseline (speedup 1.0000x reference)
import jax
import jax.numpy as jnp
from jax.experimental import pallas as pl
from jax.experimental.pallas import tpu as pltpu

SEQ_LEN = 2048
BT_SIZE = 4
D_MODEL = 4096
ROWS = SEQ_LEN * BT_SIZE
R = 1024


def _concat_kernel(x_hbm, p_ref, o_ref, sem):
    i = pl.program_id(0)
    cp = pltpu.make_async_copy(
        x_hbm.at[pl.ds(i * R, R), :], o_ref.at[:, pl.ds(0, D_MODEL)], sem
    )
    cp.start()
    o_ref[:, D_MODEL:] = p_ref[...]
    cp.wait()


def kernel(inputs, pos_table):
    x = inputs.reshape(ROWS, D_MODEL)
    out = pl.pallas_call(
        _concat_kernel,
        grid=(ROWS // R,),
        in_specs=[
            pl.BlockSpec(memory_space=pltpu.MemorySpace.HBM),
            pl.BlockSpec((R, 1), lambda i: (i % (SEQ_LEN // R), 0)),
        ],
        out_specs=pl.BlockSpec((R, D_MODEL + 1), lambda i: (i, 0)),
        out_shape=jax.ShapeDtypeStruct((ROWS, D_MODEL + 1), jnp.float32),
        scratch_shapes=[pltpu.SemaphoreType.DMA],
        compiler_params=pltpu.CompilerParams(
            dimension_semantics=("arbitrary",),
            vmem_limit_bytes=64 * 1024 * 1024,
        ),
    )(x, pos_table)
    return out.reshape(BT_SIZE, SEQ_LEN, D_MODEL + 1)
